# baseline (speedup 1.0000x reference)
import functools

import jax
import jax.numpy as jnp
from jax.experimental import pallas as pl
from jax.experimental.pallas import tpu as pltpu

N = 10000
D_IN = 128
D_OUT = 128
CHUNK_ROWS = 400
NCHUNKS = N // CHUNK_ROWS


def _gcn_kernel(x_ref, a_ref, w_ref, o_ref, s_ref, buf0, buf1, sem_ref):
    i = pl.program_id(0)

    def chunk_copy(chunk_idx, buf, sem_idx):
        return pltpu.make_async_copy(
            a_ref.at[pl.ds(chunk_idx * CHUNK_ROWS, CHUNK_ROWS), :],
            buf,
            sem_ref.at[sem_idx],
        )

    @pl.when(i == 0)
    def _bootstrap():
        chunk_copy(0, buf0, 0).start()
        chunk_copy(1, buf1, 1).start()
        s_ref[...] = jnp.dot(
            x_ref[...], w_ref[...], preferred_element_type=jnp.float32
        )

    parity = jax.lax.rem(i, 2)

    @pl.when(parity == 0)
    def _even():
        chunk_copy(i, buf0, 0).wait()
        o_ref[...] = jnp.dot(
            buf0[...], s_ref[...], preferred_element_type=jnp.float32
        )

        @pl.when(i + 2 < NCHUNKS)
        def _():
            chunk_copy(i + 2, buf0, 0).start()

    @pl.when(parity == 1)
    def _odd():
        chunk_copy(i, buf1, 1).wait()
        o_ref[...] = jnp.dot(
            buf1[...], s_ref[...], preferred_element_type=jnp.float32
        )

        @pl.when(i + 2 < NCHUNKS)
        def _():
            chunk_copy(i + 2, buf1, 1).start()


@functools.partial(jax.jit, static_argnames=())
def kernel(X, A, W):
    n, d_in = X.shape
    d_out = W.shape[1]
    return pl.pallas_call(
        _gcn_kernel,
        grid=(NCHUNKS,),
        in_specs=[
            pl.BlockSpec((n, d_in), lambda i: (0, 0)),
            pl.BlockSpec(memory_space=pltpu.MemorySpace.HBM),
            pl.BlockSpec((d_in, d_out), lambda i: (0, 0)),
        ],
        out_specs=pl.BlockSpec((CHUNK_ROWS, d_out), lambda i: (i, 0)),
        out_shape=jax.ShapeDtypeStruct((n, d_out), jnp.float32),
        scratch_shapes=[
            pltpu.VMEM((n, d_out), jnp.float32),
            pltpu.VMEM((CHUNK_ROWS, n), jnp.float32),
            pltpu.VMEM((CHUNK_ROWS, n), jnp.float32),
            pltpu.SemaphoreType.DMA((2,)),
        ],
        compiler_params=pltpu.CompilerParams(
            vmem_limit_bytes=120 * 1024 * 1024,
        ),
    )(X, A, W)


# speedup vs baseline: 1.0123x; 1.0123x over previous
import functools

import jax
import jax.numpy as jnp
from jax.experimental import pallas as pl
from jax.experimental.pallas import tpu as pltpu

N = 10000
D_IN = 128
D_OUT = 128
CHUNK_ROWS = 400
NCHUNKS = N // CHUNK_ROWS


NPIECE = 5
PIECE_ROWS = CHUNK_ROWS // NPIECE
LAST = NCHUNKS - 1


def _gcn_kernel(x_ref, a_ref, w_ref, o_ref, s_ref, buf0, buf1, sem_ref,
                tsem_ref):
    i = pl.program_id(0)

    def chunk_copy(chunk_idx, buf, sem_idx):
        return pltpu.make_async_copy(
            a_ref.at[pl.ds(chunk_idx * CHUNK_ROWS, CHUNK_ROWS), :],
            buf,
            sem_ref.at[sem_idx],
        )

    def piece_copy(p):
        return pltpu.make_async_copy(
            a_ref.at[pl.ds(LAST * CHUNK_ROWS + p * PIECE_ROWS,
                           PIECE_ROWS), :],
            buf0.at[pl.ds(p * PIECE_ROWS, PIECE_ROWS), :],
            tsem_ref.at[p],
        )

    @pl.when(i == 0)
    def _bootstrap():
        chunk_copy(0, buf0, 0).start()
        chunk_copy(1, buf1, 1).start()
        s_ref[...] = jnp.dot(
            x_ref[...], w_ref[...], preferred_element_type=jnp.float32
        )

    parity = jax.lax.rem(i, 2)

    @pl.when(jnp.logical_and(parity == 0, i < LAST))
    def _even():
        chunk_copy(i, buf0, 0).wait()
        o_ref[...] = jnp.dot(
            buf0[...], s_ref[...], preferred_element_type=jnp.float32
        )

        @pl.when(i + 2 < LAST)
        def _():
            chunk_copy(i + 2, buf0, 0).start()

        @pl.when(i + 2 == LAST)
        def _():
            for p in range(NPIECE):
                piece_copy(p).start()

    @pl.when(parity == 1)
    def _odd():
        chunk_copy(i, buf1, 1).wait()
        o_ref[...] = jnp.dot(
            buf1[...], s_ref[...], preferred_element_type=jnp.float32
        )

        @pl.when(i + 2 < LAST)
        def _():
            chunk_copy(i + 2, buf1, 1).start()

    @pl.when(i == LAST)
    def _tail():
        for p in range(NPIECE):
            piece_copy(p).wait()
            rows = pl.ds(p * PIECE_ROWS, PIECE_ROWS)
            o_ref[rows, :] = jnp.dot(
                buf0[rows, :], s_ref[...],
                preferred_element_type=jnp.float32,
            )


@functools.partial(jax.jit, static_argnames=())
def kernel(X, A, W):
    n, d_in = X.shape
    d_out = W.shape[1]
    return pl.pallas_call(
        _gcn_kernel,
        grid=(NCHUNKS,),
        in_specs=[
            pl.BlockSpec((n, d_in), lambda i: (0, 0)),
            pl.BlockSpec(memory_space=pltpu.MemorySpace.HBM),
            pl.BlockSpec((d_in, d_out), lambda i: (0, 0)),
        ],
        out_specs=pl.BlockSpec((CHUNK_ROWS, d_out), lambda i: (i, 0)),
        out_shape=jax.ShapeDtypeStruct((n, d_out), jnp.float32),
        scratch_shapes=[
            pltpu.VMEM((n, d_out), jnp.float32),
            pltpu.VMEM((CHUNK_ROWS, n), jnp.float32),
            pltpu.VMEM((CHUNK_ROWS, n), jnp.float32),
            pltpu.SemaphoreType.DMA((2,)),
            pltpu.SemaphoreType.DMA((NPIECE,)),
        ],
        compiler_params=pltpu.CompilerParams(
            vmem_limit_bytes=120 * 1024 * 1024,
        ),
    )(X, A, W)
